# baseline (speedup 1.0000x reference)
import functools

import jax
import jax.numpy as jnp
from jax import lax
from jax.experimental import pallas as pl
from jax.experimental.pallas import tpu as pltpu
from jax.experimental.pallas import tpu_sc as plsc

_N = 10000
_E = 320000
_DL = 64
_NC = 2
_NS = 16
_NW = _NC * _NS
_LN = 128
_K = 2
_CH = _K * _LN
_E2 = 327680
_NCHUNK = _E2 // (_NW * _CH)
_ROWS_W = _E2 // (_NW * _LN)
_NPAD = 10240
_SP = _NPAD // _NS
_FAST_CID = 0
_RW_FAST = 80
_RW_SLOW = 80
_IDXROWS = 16 * _RW_FAST + 15 * _RW_SLOW + _RW_FAST
_NREP = 4

_f32 = jnp.float32


def _elu(v):
    return jnp.where(v > 0, v, jnp.exp(jnp.minimum(v, 0.0)) - 1.0)


def _enc_body(x_ref, w0, b0, w1, b1, w2, b2, wa, wb, h_ref, a_ref, b_ref):
    a = _elu(jnp.dot(x_ref[...], w0[...], preferred_element_type=_f32) + b0[...])
    a = _elu(jnp.dot(a, w1[...], preferred_element_type=_f32) + b1[...])
    h = _elu(jnp.dot(a, w2[...], preferred_element_type=_f32) + b2[...])
    h_ref[...] = h
    a_ref[...] = jnp.dot(h, wa[...], preferred_element_type=_f32)
    b_ref[...] = jnp.dot(h, wb[...], preferred_element_type=_f32)


def _encoder(x, w0, b0, w1, b1, w2, b2, wa, wb):
    blk = 1000
    full = lambda s: pl.BlockSpec(s, lambda i: tuple(0 for _ in s))
    return pl.pallas_call(
        _enc_body,
        grid=(_N // blk,),
        in_specs=[
            pl.BlockSpec((blk, 128), lambda i: (i, 0)),
            full((128, 128)), full((1, 128)),
            full((128, 128)), full((1, 128)),
            full((128, _DL)), full((1, _DL)),
            full((_DL, 128)), full((_DL, 128)),
        ],
        out_specs=[
            pl.BlockSpec((blk, _DL), lambda i: (i, 0)),
            pl.BlockSpec((blk, 128), lambda i: (i, 0)),
            pl.BlockSpec((blk, 128), lambda i: (i, 0)),
        ],
        out_shape=[
            jax.ShapeDtypeStruct((_N, _DL), _f32),
            jax.ShapeDtypeStruct((_N, 128), _f32),
            jax.ShapeDtypeStruct((_N, 128), _f32),
        ],
    )(x, w0, b0, w1, b1, w2, b2, wa, wb)


_mesh = plsc.VectorSubcoreMesh(core_axis_name="c", subcore_axis_name="s")


@functools.partial(
    pl.kernel,
    out_type=jax.ShapeDtypeStruct((_E2, 128), _f32),
    mesh=_mesh,
    scratch_types=[
        pltpu.VMEM((_RW_FAST, _LN), jnp.int32),
        pltpu.VMEM((_RW_FAST, _LN), jnp.int32),
        pltpu.VMEM((4, _LN, 128), _f32),
        pltpu.SemaphoreType.DMA, pltpu.SemaphoreType.DMA,
        pltpu.SemaphoreType.DMA, pltpu.SemaphoreType.DMA,
        pltpu.SemaphoreType.DMA, pltpu.SemaphoreType.DMA,
        pltpu.SemaphoreType.DMA, pltpu.SemaphoreType.DMA,
    ],
)
def _gather_k(t_hbm, src_hbm, dst_hbm,
              pre_hbm,
              idx_s, idx_d, buf, g0, g1, g2, g3, w0, w1, w2, w3):
    cid = lax.axis_index("c")
    sid = lax.axis_index("s")
    fast = cid == _FAST_CID
    nrows = jnp.where(fast, _RW_FAST, _RW_SLOW)
    r0 = jnp.where(fast, sid * _RW_FAST,
                   16 * _RW_FAST + sid * _RW_SLOW)
    gsem = (g0, g1, g2, g3)
    wsem = (w0, w1, w2, w3)

    pltpu.sync_copy(src_hbm.at[pl.ds(r0, _RW_FAST)], idx_s)
    pltpu.sync_copy(dst_hbm.at[pl.ds(r0, _RW_FAST)], idx_d)

    def fire_a(t, b):
        pltpu.async_copy(t_hbm.at[idx_d.at[t]], buf.at[b], gsem[b])

    def fire_b(t, b):
        pltpu.async_copy(t_hbm.at[idx_s.at[t]], buf.at[b], gsem[b], add=True)

    def wait_g(b):
        pltpu.make_async_copy(t_hbm.at[pl.ds(0, _LN)], buf.at[b],
                              gsem[b]).wait()

    def fire_w(t, b):
        e = (r0 + t) * _LN
        pltpu.async_copy(buf.at[b], pre_hbm.at[pl.ds(e, _LN)], wsem[b])

    def wait_w(b):
        pltpu.make_async_copy(buf.at[b], pre_hbm.at[pl.ds(0, _LN)],
                              wsem[b]).wait()

    fire_a(0, 0)
    fire_a(1, 1)

    @pl.loop(0, _RW_FAST, step=4)
    def _(t):
        for j in range(4):
            k = j
            kp = (j + 2) % 4
            row = t + j
            live = row < nrows

            @pl.when(live)
            def _():
                wait_g(k)
                fire_b(row, k)

            @pl.when(jnp.logical_and(row + 2 < nrows, row >= 2))
            def _():
                wait_w(kp)

            @pl.when(row + 2 < nrows)
            def _():
                fire_a(row + 2, kp)

            @pl.when(live)
            def _():
                wait_g(k)
                fire_w(row, k)

    for k in range(4):
        wait_w(k)


def _edge_body(pre_ref, ea_ref, w0c, b0, w1p, b1p, m_ref):
    i = pl.program_id(0)
    blk = pre_ref.shape[0]
    pre = (pre_ref[...]
           + jnp.dot(ea_ref[...], w0c[...], preferred_element_type=_f32)
           + b0[...])
    m2 = jnp.dot(_elu(pre), w1p[...], preferred_element_type=_f32) + b1p[...]
    eid = i * blk + lax.broadcasted_iota(jnp.int32, (blk, 1), 0)
    m_ref[...] = jnp.where(eid < _E, m2, 0.0)


def _edge_mlp(pre0, ea, w0c, b0, w1p, b1p):
    blk = 4096
    full = lambda s: pl.BlockSpec(s, lambda i: tuple(0 for _ in s))
    return pl.pallas_call(
        _edge_body,
        grid=(_E2 // blk,),
        in_specs=[
            pl.BlockSpec((blk, 128), lambda i: (i, 0)),
            pl.BlockSpec((blk, 16), lambda i: (i, 0)),
            full((16, 128)), full((1, 128)),
            full((128, 128)), full((1, 128)),
        ],
        out_specs=pl.BlockSpec((blk, 128), lambda i: (i, 0)),
        out_shape=jax.ShapeDtypeStruct((_E2, 128), _f32),
    )(pre0, ea, w0c, b0, w1p, b1p)


@functools.partial(
    pl.kernel,
    out_type=jax.ShapeDtypeStruct((_NC, _NPAD, 128), _f32),
    mesh=_mesh,
    scratch_types=[
        pltpu.VMEM((_ROWS_W, _LN), jnp.int32),
        pltpu.VMEM((2, _LN, 128), _f32),
        pltpu.VMEM_SHARED((_NPAD, 128), _f32),
        pltpu.SemaphoreType.DMA, pltpu.SemaphoreType.DMA,
    ],
)
def _scatter_k(m_hbm, dst_hbm, z_hbm, s_hbm, idx_d, m_v, acc_sh, r0s, r1s):
    cid = lax.axis_index("c")
    sid = lax.axis_index("s")
    wid = sid * _NC + cid
    r0 = wid * _ROWS_W
    rsem = (r0s, r1s)

    pltpu.sync_copy(z_hbm.at[pl.ds(sid * _SP, _SP)],
                    acc_sh.at[pl.ds(sid * _SP, _SP)])
    pltpu.sync_copy(dst_hbm.at[pl.ds(r0, _ROWS_W)], idx_d)
    plsc.subcore_barrier()

    def fire_r(t, b):
        pltpu.async_copy(m_hbm.at[pl.ds((r0 + t) * _LN, _LN)], m_v.at[b],
                         rsem[b])

    def wait_r(b):
        pltpu.make_async_copy(m_hbm.at[pl.ds(0, _LN)], m_v.at[b],
                              rsem[b]).wait()

    fire_r(0, 0)

    @pl.loop(0, _ROWS_W, step=2)
    def _(t):
        fire_r(t + 1, 1)
        wait_r(0)
        pltpu.sync_copy(m_v.at[0], acc_sh.at[idx_d.at[t]], add=True)

        @pl.when(t + 2 < _ROWS_W)
        def _():
            fire_r(t + 2, 0)
        wait_r(1)
        pltpu.sync_copy(m_v.at[1], acc_sh.at[idx_d.at[t + 1]], add=True)

    plsc.subcore_barrier()
    pltpu.sync_copy(acc_sh.at[pl.ds(sid * _SP, _SP)],
                    s_hbm.at[cid, pl.ds(sid * _SP, _SP)])


def _fin_body(h_ref, s_ref, gw0a, gw0b, gb0, gw1, gb1,
              dw0, db0, dw1, db1, dw2, db2, y_ref):
    s = s_ref[0] + s_ref[1]
    aggr = s[:, :_DL] / jnp.maximum(s[:, _DL:_DL + 1], 1.0)
    g = _elu(jnp.dot(h_ref[...], gw0a[...], preferred_element_type=_f32)
             + jnp.dot(aggr, gw0b[...], preferred_element_type=_f32)
             + gb0[...])
    g = _elu(jnp.dot(g, gw1[...], preferred_element_type=_f32) + gb1[...])
    d = _elu(jnp.dot(g, dw0[...], preferred_element_type=_f32) + db0[...])
    d = _elu(jnp.dot(d, dw1[...], preferred_element_type=_f32) + db1[...])
    y_ref[...] = jnp.dot(d, dw2[...], preferred_element_type=_f32) + db2[...]


def _final(h, s_p, gw0a, gw0b, gb0, gw1, gb1, dw0, db0, dw1, db1, dw2, db2):
    blk = 1000
    full = lambda s: pl.BlockSpec(s, lambda i: tuple(0 for _ in s))
    return pl.pallas_call(
        _fin_body,
        grid=(_N // blk,),
        in_specs=[
            pl.BlockSpec((blk, _DL), lambda i: (i, 0)),
            pl.BlockSpec((_NC, blk, 128), lambda i: (0, i, 0)),
            full((_DL, 128)), full((_DL, 128)), full((1, 128)),
            full((128, _DL)), full((1, _DL)),
            full((_DL, 128)), full((1, 128)),
            full((128, 128)), full((1, 128)),
            full((128, 3)), full((1, 3)),
        ],
        out_specs=pl.BlockSpec((blk, 3), lambda i: (i, 0)),
        out_shape=jax.ShapeDtypeStruct((_N, 3), _f32),
    )(h, s_p, gw0a, gw0b, gb0, gw1, gb1, dw0, db0, dw1, db1, dw2, db2)


def kernel(x, edge_index, edge_attr,
           enc_W0, enc_b0, enc_W1, enc_b1, enc_W2, enc_b2,
           phi_W0, phi_b0, phi_W1, phi_b1,
           gam_W0, gam_b0, gam_W1, gam_b1,
           dec_W0, dec_b0, dec_W1, dec_b1, dec_W2, dec_b2):
    ipad = _IDXROWS * _LN - _E
    src = jnp.concatenate([edge_index[0], jnp.zeros((ipad,), jnp.int32)])
    dst = jnp.concatenate([edge_index[1], jnp.zeros((ipad,), jnp.int32)])
    src2 = src.reshape(_IDXROWS, _LN)
    dst2 = dst.reshape(_IDXROWS, _LN)
    ea2 = jnp.concatenate([edge_attr, jnp.zeros((_E2 - _E, 16), _f32)],
                          axis=0)

    row_ids = jnp.arange(_IDXROWS, dtype=jnp.int32)
    wrk = jnp.minimum(row_ids // _RW_FAST, _NW - 1)
    rep = 2 * (wrk // _NS) + ((wrk % _NS) % 2)
    aoff = (rep * 2 * _N)[:, None]
    boff = (rep * 2 * _N + _N)[:, None]
    dst2g = dst2 + aoff
    src2g = src2 + boff

    h, A, B = _encoder(x, enc_W0, enc_b0.reshape(1, -1),
                       enc_W1, enc_b1.reshape(1, -1),
                       enc_W2, enc_b2.reshape(1, -1),
                       phi_W0[:_DL], phi_W0[_DL:2 * _DL])

    tab = jnp.concatenate([A, B] * _NREP, axis=0)
    pre0 = _gather_k(tab, src2g, dst2g)

    w1p = jnp.concatenate([phi_W1, jnp.zeros((128, 128 - _DL), _f32)], axis=1)
    b1p = jnp.concatenate(
        [phi_b1, jnp.ones((1,), _f32), jnp.zeros((128 - _DL - 1,), _f32)])
    m2 = _edge_mlp(pre0, ea2, phi_W0[2 * _DL:], phi_b0.reshape(1, -1),
                   w1p, b1p.reshape(1, -1))

    z = jnp.zeros((_NPAD, 128), _f32)
    s_p = _scatter_k(m2, dst2, z)

    return _final(h, s_p,
                  gam_W0[:_DL], gam_W0[_DL:], gam_b0.reshape(1, -1),
                  gam_W1, gam_b1.reshape(1, -1),
                  dec_W0, dec_b0.reshape(1, -1),
                  dec_W1, dec_b1.reshape(1, -1),
                  dec_W2, dec_b2.reshape(1, -1))


# speedup vs baseline: 1.1107x; 1.1107x over previous
import functools

import jax
import jax.numpy as jnp
from jax import lax
from jax.experimental import pallas as pl
from jax.experimental.pallas import tpu as pltpu
from jax.experimental.pallas import tpu_sc as plsc

_N = 10000
_E = 320000
_DL = 64
_NC = 2
_NS = 16
_NW = _NC * _NS
_LN = 128
_K = 2
_CH = _K * _LN
_E2 = 327680
_NCHUNK = _E2 // (_NW * _CH)
_ROWS_W = _E2 // (_NW * _LN)
_NPAD = 10240
_SP = _NPAD // _NS
_FAST_CID = 0
_RW_FAST = 80
_RW_SLOW = 80
_IDXROWS = 16 * _RW_FAST + 15 * _RW_SLOW + _RW_FAST

_f32 = jnp.float32


def _elu(v):
    return jnp.where(v > 0, v, jnp.exp(jnp.minimum(v, 0.0)) - 1.0)


def _enc_body(x_ref, w0, b0, w1, b1, w2, b2, wa, wb, h_ref, a_ref, b_ref):
    a = _elu(jnp.dot(x_ref[...], w0[...], preferred_element_type=_f32) + b0[...])
    a = _elu(jnp.dot(a, w1[...], preferred_element_type=_f32) + b1[...])
    h = _elu(jnp.dot(a, w2[...], preferred_element_type=_f32) + b2[...])
    h_ref[...] = h
    a_ref[...] = jnp.dot(h, wa[...], preferred_element_type=_f32)
    b_ref[...] = jnp.dot(h, wb[...], preferred_element_type=_f32)


def _encoder(x, w0, b0, w1, b1, w2, b2, wa, wb):
    blk = 1000
    full = lambda s: pl.BlockSpec(s, lambda i: tuple(0 for _ in s))
    return pl.pallas_call(
        _enc_body,
        grid=(_N // blk,),
        in_specs=[
            pl.BlockSpec((blk, 128), lambda i: (i, 0)),
            full((128, 128)), full((1, 128)),
            full((128, 128)), full((1, 128)),
            full((128, _DL)), full((1, _DL)),
            full((_DL, 128)), full((_DL, 128)),
        ],
        out_specs=[
            pl.BlockSpec((blk, _DL), lambda i: (i, 0)),
            pl.BlockSpec((blk, 128), lambda i: (i, 0)),
            pl.BlockSpec((blk, 128), lambda i: (i, 0)),
        ],
        out_shape=[
            jax.ShapeDtypeStruct((_N, _DL), _f32),
            jax.ShapeDtypeStruct((_N, 128), _f32),
            jax.ShapeDtypeStruct((_N, 128), _f32),
        ],
    )(x, w0, b0, w1, b1, w2, b2, wa, wb)


_mesh = plsc.VectorSubcoreMesh(core_axis_name="c", subcore_axis_name="s")


def _make_gather(rw):
    @functools.partial(
        pl.kernel,
        out_type=jax.ShapeDtypeStruct((rw * _NW * _LN, 128), _f32),
        mesh=_mesh,
        scratch_types=[
            pltpu.VMEM((rw, _LN), jnp.int32),
            pltpu.VMEM((rw, _LN), jnp.int32),
            pltpu.VMEM((4, _LN, 128), _f32),
            pltpu.SemaphoreType.DMA, pltpu.SemaphoreType.DMA,
            pltpu.SemaphoreType.DMA, pltpu.SemaphoreType.DMA,
            pltpu.SemaphoreType.DMA, pltpu.SemaphoreType.DMA,
            pltpu.SemaphoreType.DMA, pltpu.SemaphoreType.DMA,
        ],
    )
    def gk(a_hbm, b_hbm, src_hbm, dst_hbm,
           pre_hbm,
           idx_s, idx_d, buf, g0, g1, g2, g3, w0, w1, w2, w3):
        cid = lax.axis_index("c")
        sid = lax.axis_index("s")
        r0 = (cid * _NS + sid) * rw
        gsem = (g0, g1, g2, g3)
        wsem = (w0, w1, w2, w3)

        pltpu.sync_copy(src_hbm.at[pl.ds(r0, rw)], idx_s)
        pltpu.sync_copy(dst_hbm.at[pl.ds(r0, rw)], idx_d)

        def fire_a(t, b):
            pltpu.async_copy(a_hbm.at[idx_d.at[t]], buf.at[b], gsem[b])

        def fire_b(t, b):
            pltpu.async_copy(b_hbm.at[idx_s.at[t]], buf.at[b], gsem[b],
                             add=True)

        def wait_g(b):
            pltpu.make_async_copy(a_hbm.at[pl.ds(0, _LN)], buf.at[b],
                                  gsem[b]).wait()

        def fire_w(t, b):
            e = (r0 + t) * _LN
            pltpu.async_copy(buf.at[b], pre_hbm.at[pl.ds(e, _LN)], wsem[b])

        def wait_w(b):
            pltpu.make_async_copy(buf.at[b], pre_hbm.at[pl.ds(0, _LN)],
                                  wsem[b]).wait()

        fire_a(0, 0)
        fire_a(1, 1)

        @pl.loop(0, rw, step=4)
        def _(t):
            for j in range(4):
                k = j
                kp = (j + 2) % 4
                row = t + j
                wait_g(k)
                fire_b(row, k)

                @pl.when(jnp.logical_and(row + 2 < rw, row >= 2))
                def _():
                    wait_w(kp)

                @pl.when(row + 2 < rw)
                def _():
                    fire_a(row + 2, kp)
                wait_g(k)
                fire_w(row, k)

        for k in range(4):
            wait_w(k)

    return gk


_gather_half = _make_gather(_ROWS_W // 2)


def _edge_body(base, pre_ref, ea_ref, w0c, b0, w1p, b1p, m_ref):
    i = pl.program_id(0)
    blk = pre_ref.shape[0]
    pre = (pre_ref[...]
           + jnp.dot(ea_ref[...], w0c[...], preferred_element_type=_f32)
           + b0[...])
    m2 = jnp.dot(_elu(pre), w1p[...], preferred_element_type=_f32) + b1p[...]
    eid = base + i * blk + lax.broadcasted_iota(jnp.int32, (blk, 1), 0)
    m_ref[...] = jnp.where(eid < _E, m2, 0.0)


def _edge_mlp(pre0, ea, w0c, b0, w1p, b1p, base):
    blk = 4096
    n = pre0.shape[0]
    full = lambda s: pl.BlockSpec(s, lambda i: tuple(0 for _ in s))
    return pl.pallas_call(
        functools.partial(_edge_body, base),
        grid=(n // blk,),
        in_specs=[
            pl.BlockSpec((blk, 128), lambda i: (i, 0)),
            pl.BlockSpec((blk, 16), lambda i: (i, 0)),
            full((16, 128)), full((1, 128)),
            full((128, 128)), full((1, 128)),
        ],
        out_specs=pl.BlockSpec((blk, 128), lambda i: (i, 0)),
        out_shape=jax.ShapeDtypeStruct((n, 128), _f32),
    )(pre0, ea, w0c, b0, w1p, b1p)


def _make_scatter(rw):
    @functools.partial(
        pl.kernel,
        out_type=jax.ShapeDtypeStruct((_NC, _NPAD, 128), _f32),
        mesh=_mesh,
        scratch_types=[
            pltpu.VMEM((rw, _LN), jnp.int32),
            pltpu.VMEM((2, _LN, 128), _f32),
            pltpu.VMEM_SHARED((_NPAD, 128), _f32),
            pltpu.SemaphoreType.DMA, pltpu.SemaphoreType.DMA,
        ],
    )
    def sk(m_hbm, dst_hbm, z_hbm, s_hbm, idx_d, m_v, acc_sh, r0s, r1s):
        cid = lax.axis_index("c")
        sid = lax.axis_index("s")
        wid = sid * _NC + cid
        r0 = wid * rw
        rsem = (r0s, r1s)

        pltpu.sync_copy(z_hbm.at[pl.ds(sid * _SP, _SP)],
                        acc_sh.at[pl.ds(sid * _SP, _SP)])
        pltpu.sync_copy(dst_hbm.at[pl.ds(r0, rw)], idx_d)
        plsc.subcore_barrier()

        def fire_r(t, b):
            pltpu.async_copy(m_hbm.at[pl.ds((r0 + t) * _LN, _LN)], m_v.at[b],
                             rsem[b])

        def wait_r(b):
            pltpu.make_async_copy(m_hbm.at[pl.ds(0, _LN)], m_v.at[b],
                                  rsem[b]).wait()

        fire_r(0, 0)

        @pl.loop(0, rw, step=2)
        def _(t):
            fire_r(t + 1, 1)
            wait_r(0)
            pltpu.sync_copy(m_v.at[0], acc_sh.at[idx_d.at[t]], add=True)

            @pl.when(t + 2 < rw)
            def _():
                fire_r(t + 2, 0)
            wait_r(1)
            pltpu.sync_copy(m_v.at[1], acc_sh.at[idx_d.at[t + 1]], add=True)

        plsc.subcore_barrier()
        pltpu.sync_copy(acc_sh.at[pl.ds(sid * _SP, _SP)],
                        s_hbm.at[cid, pl.ds(sid * _SP, _SP)])

    return sk


_scatter_half = _make_scatter(_ROWS_W // 2)


def _fin_body(h_ref, s_ref, s2_ref, gw0a, gw0b, gb0, gw1, gb1,
              dw0, db0, dw1, db1, dw2, db2, y_ref):
    s = s_ref[0] + s_ref[1] + s2_ref[0] + s2_ref[1]
    aggr = s[:, :_DL] / jnp.maximum(s[:, _DL:_DL + 1], 1.0)
    g = _elu(jnp.dot(h_ref[...], gw0a[...], preferred_element_type=_f32)
             + jnp.dot(aggr, gw0b[...], preferred_element_type=_f32)
             + gb0[...])
    g = _elu(jnp.dot(g, gw1[...], preferred_element_type=_f32) + gb1[...])
    d = _elu(jnp.dot(g, dw0[...], preferred_element_type=_f32) + db0[...])
    d = _elu(jnp.dot(d, dw1[...], preferred_element_type=_f32) + db1[...])
    y_ref[...] = jnp.dot(d, dw2[...], preferred_element_type=_f32) + db2[...]


def _final(h, s_p, s_p2, gw0a, gw0b, gb0, gw1, gb1,
           dw0, db0, dw1, db1, dw2, db2):
    blk = 1000
    full = lambda s: pl.BlockSpec(s, lambda i: tuple(0 for _ in s))
    return pl.pallas_call(
        _fin_body,
        grid=(_N // blk,),
        in_specs=[
            pl.BlockSpec((blk, _DL), lambda i: (i, 0)),
            pl.BlockSpec((_NC, blk, 128), lambda i: (0, i, 0)),
            pl.BlockSpec((_NC, blk, 128), lambda i: (0, i, 0)),
            full((_DL, 128)), full((_DL, 128)), full((1, 128)),
            full((128, _DL)), full((1, _DL)),
            full((_DL, 128)), full((1, 128)),
            full((128, 128)), full((1, 128)),
            full((128, 3)), full((1, 3)),
        ],
        out_specs=pl.BlockSpec((blk, 3), lambda i: (i, 0)),
        out_shape=jax.ShapeDtypeStruct((_N, 3), _f32),
    )(h, s_p, s_p2, gw0a, gw0b, gb0, gw1, gb1,
      dw0, db0, dw1, db1, dw2, db2)


def kernel(x, edge_index, edge_attr,
           enc_W0, enc_b0, enc_W1, enc_b1, enc_W2, enc_b2,
           phi_W0, phi_b0, phi_W1, phi_b1,
           gam_W0, gam_b0, gam_W1, gam_b1,
           dec_W0, dec_b0, dec_W1, dec_b1, dec_W2, dec_b2):
    ipad = _IDXROWS * _LN - _E
    src = jnp.concatenate([edge_index[0], jnp.zeros((ipad,), jnp.int32)])
    dst = jnp.concatenate([edge_index[1], jnp.zeros((ipad,), jnp.int32)])
    src2 = src.reshape(_IDXROWS, _LN)
    dst2 = dst.reshape(_IDXROWS, _LN)
    ea2 = jnp.concatenate([edge_attr, jnp.zeros((_E2 - _E, 16), _f32)],
                          axis=0)

    h, A, B = _encoder(x, enc_W0, enc_b0.reshape(1, -1),
                       enc_W1, enc_b1.reshape(1, -1),
                       enc_W2, enc_b2.reshape(1, -1),
                       phi_W0[:_DL], phi_W0[_DL:2 * _DL])

    w1p = jnp.concatenate([phi_W1, jnp.zeros((128, 128 - _DL), _f32)], axis=1)
    b1p = jnp.concatenate(
        [phi_b1, jnp.ones((1,), _f32), jnp.zeros((128 - _DL - 1,), _f32)])
    w0c = phi_W0[2 * _DL:]
    b0r = phi_b0.reshape(1, -1)
    b1r = b1p.reshape(1, -1)
    z = jnp.zeros((_NPAD, 128), _f32)

    eh = _E2 // 2
    rh = _IDXROWS // 2
    pre_a = _gather_half(A, B, src2[:rh], dst2[:rh])
    pre_b = _gather_half(A, B, src2[rh:], dst2[rh:])
    m_a = _edge_mlp(pre_a, ea2[:eh], w0c, b0r, w1p, b1r, 0)
    s_pa = _scatter_half(m_a, dst2[:rh], z)
    m_b = _edge_mlp(pre_b, ea2[eh:], w0c, b0r, w1p, b1r, eh)
    s_pb = _scatter_half(m_b, dst2[rh:], z)

    return _final(h, s_pa, s_pb,
                  gam_W0[:_DL], gam_W0[_DL:], gam_b0.reshape(1, -1),
                  gam_W1, gam_b1.reshape(1, -1),
                  dec_W0, dec_b0.reshape(1, -1),
                  dec_W1, dec_b1.reshape(1, -1),
                  dec_W2, dec_b2.reshape(1, -1))


# speedup vs baseline: 1.1107x; 1.0000x over previous
import functools

import jax
import jax.numpy as jnp
from jax import lax
from jax.experimental import pallas as pl
from jax.experimental.pallas import tpu as pltpu
from jax.experimental.pallas import tpu_sc as plsc

_N = 10000
_E = 320000
_DL = 64
_NC = 2
_NS = 16
_NW = _NC * _NS
_LN = 128
_K = 2
_CH = _K * _LN
_E2 = 327680
_NCHUNK = _E2 // (_NW * _CH)
_ROWS_W = _E2 // (_NW * _LN)
_NPAD = 10240
_SP = _NPAD // _NS
_FAST_CID = 0
_RW_FAST = 80
_RW_SLOW = 80
_IDXROWS = 16 * _RW_FAST + 15 * _RW_SLOW + _RW_FAST

_f32 = jnp.float32


def _elu(v):
    return jnp.where(v > 0, v, jnp.exp(jnp.minimum(v, 0.0)) - 1.0)


def _enc_body(x_ref, w0, b0, w1, b1, w2, b2, wa, wb, h_ref, a_ref, b_ref):
    a = _elu(jnp.dot(x_ref[...], w0[...], preferred_element_type=_f32) + b0[...])
    a = _elu(jnp.dot(a, w1[...], preferred_element_type=_f32) + b1[...])
    h = _elu(jnp.dot(a, w2[...], preferred_element_type=_f32) + b2[...])
    h_ref[...] = h
    a_ref[...] = jnp.dot(h, wa[...], preferred_element_type=_f32)
    b_ref[...] = jnp.dot(h, wb[...], preferred_element_type=_f32)


def _encoder(x, w0, b0, w1, b1, w2, b2, wa, wb):
    blk = 1000
    full = lambda s: pl.BlockSpec(s, lambda i: tuple(0 for _ in s))
    return pl.pallas_call(
        _enc_body,
        grid=(_N // blk,),
        in_specs=[
            pl.BlockSpec((blk, 128), lambda i: (i, 0)),
            full((128, 128)), full((1, 128)),
            full((128, 128)), full((1, 128)),
            full((128, _DL)), full((1, _DL)),
            full((_DL, 128)), full((_DL, 128)),
        ],
        out_specs=[
            pl.BlockSpec((blk, _DL), lambda i: (i, 0)),
            pl.BlockSpec((blk, 128), lambda i: (i, 0)),
            pl.BlockSpec((blk, 128), lambda i: (i, 0)),
        ],
        out_shape=[
            jax.ShapeDtypeStruct((_N, _DL), _f32),
            jax.ShapeDtypeStruct((_N, 128), _f32),
            jax.ShapeDtypeStruct((_N, 128), _f32),
        ],
    )(x, w0, b0, w1, b1, w2, b2, wa, wb)


_mesh = plsc.VectorSubcoreMesh(core_axis_name="c", subcore_axis_name="s")


def _make_gather(rw):
    @functools.partial(
        pl.kernel,
        out_type=jax.ShapeDtypeStruct((rw * _NW * _LN, 128), _f32),
        mesh=_mesh,
        scratch_types=[
            pltpu.VMEM((rw, _LN), jnp.int32),
            pltpu.VMEM((rw, _LN), jnp.int32),
            pltpu.VMEM((4, _LN, 128), _f32),
            pltpu.SemaphoreType.DMA, pltpu.SemaphoreType.DMA,
            pltpu.SemaphoreType.DMA, pltpu.SemaphoreType.DMA,
            pltpu.SemaphoreType.DMA, pltpu.SemaphoreType.DMA,
            pltpu.SemaphoreType.DMA, pltpu.SemaphoreType.DMA,
        ],
    )
    def gk(a_hbm, b_hbm, src_hbm, dst_hbm,
           pre_hbm,
           idx_s, idx_d, buf, g0, g1, g2, g3, w0, w1, w2, w3):
        cid = lax.axis_index("c")
        sid = lax.axis_index("s")
        r0 = (cid * _NS + sid) * rw
        gsem = (g0, g1, g2, g3)
        wsem = (w0, w1, w2, w3)

        pltpu.sync_copy(src_hbm.at[pl.ds(r0, rw)], idx_s)
        pltpu.sync_copy(dst_hbm.at[pl.ds(r0, rw)], idx_d)

        def fire_a(t, b):
            pltpu.async_copy(a_hbm.at[idx_d.at[t]], buf.at[b], gsem[b])

        def fire_b(t, b):
            pltpu.async_copy(b_hbm.at[idx_s.at[t]], buf.at[b], gsem[b],
                             add=True)

        def wait_g(b):
            pltpu.make_async_copy(a_hbm.at[pl.ds(0, _LN)], buf.at[b],
                                  gsem[b]).wait()

        def fire_w(t, b):
            e = (r0 + t) * _LN
            pltpu.async_copy(buf.at[b], pre_hbm.at[pl.ds(e, _LN)], wsem[b])

        def wait_w(b):
            pltpu.make_async_copy(buf.at[b], pre_hbm.at[pl.ds(0, _LN)],
                                  wsem[b]).wait()

        fire_a(0, 0)
        fire_a(1, 1)

        @pl.loop(0, rw, step=4)
        def _(t):
            for j in range(4):
                k = j
                kp = (j + 2) % 4
                row = t + j
                wait_g(k)
                fire_b(row, k)

                @pl.when(jnp.logical_and(row + 2 < rw, row >= 2))
                def _():
                    wait_w(kp)

                @pl.when(row + 2 < rw)
                def _():
                    fire_a(row + 2, kp)
                wait_g(k)
                fire_w(row, k)

        for k in range(4):
            wait_w(k)

    return gk


_gather_half = _make_gather(_ROWS_W // 2)


def _edge_body(base, pre_ref, ea_ref, w0c, b0, w1p, b1p, m_ref):
    i = pl.program_id(0)
    blk = pre_ref.shape[0]
    pre = (pre_ref[...]
           + jnp.dot(ea_ref[...], w0c[...], preferred_element_type=_f32)
           + b0[...])
    m2 = jnp.dot(_elu(pre), w1p[...], preferred_element_type=_f32) + b1p[...]
    eid = base + i * blk + lax.broadcasted_iota(jnp.int32, (blk, 1), 0)
    m_ref[...] = jnp.where(eid < _E, m2, 0.0)


def _edge_mlp(pre0, ea, w0c, b0, w1p, b1p, base):
    blk = 4096
    n = pre0.shape[0]
    full = lambda s: pl.BlockSpec(s, lambda i: tuple(0 for _ in s))
    return pl.pallas_call(
        functools.partial(_edge_body, base),
        grid=(n // blk,),
        in_specs=[
            pl.BlockSpec((blk, 128), lambda i: (i, 0)),
            pl.BlockSpec((blk, 16), lambda i: (i, 0)),
            full((16, 128)), full((1, 128)),
            full((128, 128)), full((1, 128)),
        ],
        out_specs=pl.BlockSpec((blk, 128), lambda i: (i, 0)),
        out_shape=jax.ShapeDtypeStruct((n, 128), _f32),
    )(pre0, ea, w0c, b0, w1p, b1p)


def _make_scatter(rw):
    @functools.partial(
        pl.kernel,
        out_type=jax.ShapeDtypeStruct((_NC, _NPAD, 128), _f32),
        mesh=_mesh,
        scratch_types=[
            pltpu.VMEM((rw, _LN), jnp.int32),
            pltpu.VMEM((2, _LN, 128), _f32),
            pltpu.VMEM_SHARED((_NPAD, 128), _f32),
            pltpu.SemaphoreType.DMA, pltpu.SemaphoreType.DMA,
        ],
    )
    def sk(m_hbm, dst_hbm, z_hbm, s_hbm, idx_d, m_v, acc_sh, r0s, r1s):
        cid = lax.axis_index("c")
        sid = lax.axis_index("s")
        wid = sid * _NC + cid
        r0 = wid * rw
        rsem = (r0s, r1s)

        pltpu.sync_copy(z_hbm.at[pl.ds(sid * _SP, _SP)],
                        acc_sh.at[pl.ds(sid * _SP, _SP)])
        pltpu.sync_copy(dst_hbm.at[pl.ds(r0, rw)], idx_d)
        plsc.subcore_barrier()

        def fire_r(t, b):
            pltpu.async_copy(m_hbm.at[pl.ds((r0 + t) * _LN, _LN)], m_v.at[b],
                             rsem[b])

        def wait_r(b):
            pltpu.make_async_copy(m_hbm.at[pl.ds(0, _LN)], m_v.at[b],
                                  rsem[b]).wait()

        fire_r(0, 0)

        @pl.loop(0, rw, step=2)
        def _(t):
            fire_r(t + 1, 1)
            wait_r(0)
            pltpu.sync_copy(m_v.at[0], acc_sh.at[idx_d.at[t]], add=True)

            @pl.when(t + 2 < rw)
            def _():
                fire_r(t + 2, 0)
            wait_r(1)
            pltpu.sync_copy(m_v.at[1], acc_sh.at[idx_d.at[t + 1]], add=True)

        plsc.subcore_barrier()
        pltpu.sync_copy(acc_sh.at[pl.ds(sid * _SP, _SP)],
                        s_hbm.at[cid, pl.ds(sid * _SP, _SP)])

    return sk


_scatter_half = _make_scatter(_ROWS_W // 2)


def _fin_body(h_ref, s_ref, s2_ref, gw0a, gw0b, gb0, gw1, gb1,
              dw0, db0, dw1, db1, dw2, db2, y_ref):
    s = s_ref[0] + s_ref[1] + s2_ref[0] + s2_ref[1]
    aggr = s[:, :_DL] / jnp.maximum(s[:, _DL:_DL + 1], 1.0)
    g = _elu(jnp.dot(h_ref[...], gw0a[...], preferred_element_type=_f32)
             + jnp.dot(aggr, gw0b[...], preferred_element_type=_f32)
             + gb0[...])
    g = _elu(jnp.dot(g, gw1[...], preferred_element_type=_f32) + gb1[...])
    d = _elu(jnp.dot(g, dw0[...], preferred_element_type=_f32) + db0[...])
    d = _elu(jnp.dot(d, dw1[...], preferred_element_type=_f32) + db1[...])
    y_ref[...] = jnp.dot(d, dw2[...], preferred_element_type=_f32) + db2[...]


def _final(h, s_p, s_p2, gw0a, gw0b, gb0, gw1, gb1,
           dw0, db0, dw1, db1, dw2, db2):
    blk = 1000
    full = lambda s: pl.BlockSpec(s, lambda i: tuple(0 for _ in s))
    return pl.pallas_call(
        _fin_body,
        grid=(_N // blk,),
        in_specs=[
            pl.BlockSpec((blk, _DL), lambda i: (i, 0)),
            pl.BlockSpec((_NC, blk, 128), lambda i: (0, i, 0)),
            pl.BlockSpec((_NC, blk, 128), lambda i: (0, i, 0)),
            full((_DL, 128)), full((_DL, 128)), full((1, 128)),
            full((128, _DL)), full((1, _DL)),
            full((_DL, 128)), full((1, 128)),
            full((128, 128)), full((1, 128)),
            full((128, 3)), full((1, 3)),
        ],
        out_specs=pl.BlockSpec((blk, 3), lambda i: (i, 0)),
        out_shape=jax.ShapeDtypeStruct((_N, 3), _f32),
    )(h, s_p, s_p2, gw0a, gw0b, gb0, gw1, gb1,
      dw0, db0, dw1, db1, dw2, db2)


def kernel(x, edge_index, edge_attr,
           enc_W0, enc_b0, enc_W1, enc_b1, enc_W2, enc_b2,
           phi_W0, phi_b0, phi_W1, phi_b1,
           gam_W0, gam_b0, gam_W1, gam_b1,
           dec_W0, dec_b0, dec_W1, dec_b1, dec_W2, dec_b2):
    ipad = _IDXROWS * _LN - _E
    src = jnp.concatenate([edge_index[0], jnp.zeros((ipad,), jnp.int32)])
    dst = jnp.concatenate([edge_index[1], jnp.zeros((ipad,), jnp.int32)])
    src2 = src.reshape(_IDXROWS, _LN)
    dst2 = dst.reshape(_IDXROWS, _LN)
    ea2 = jnp.concatenate([edge_attr, jnp.zeros((_E2 - _E, 16), _f32)],
                          axis=0)

    h, A, B = _encoder(x, enc_W0, enc_b0.reshape(1, -1),
                       enc_W1, enc_b1.reshape(1, -1),
                       enc_W2, enc_b2.reshape(1, -1),
                       phi_W0[:_DL], phi_W0[_DL:2 * _DL])

    w1p = jnp.concatenate([phi_W1, jnp.zeros((128, 128 - _DL), _f32)], axis=1)
    b1p = jnp.concatenate(
        [phi_b1, jnp.ones((1,), _f32), jnp.zeros((128 - _DL - 1,), _f32)])
    w0c = phi_W0[2 * _DL:]
    b0r = phi_b0.reshape(1, -1)
    b1r = b1p.reshape(1, -1)
    z = jnp.zeros((_NPAD, 128), _f32)

    eh = _E2 // 2
    rh = _IDXROWS // 2
    pre_b = _gather_half(A, B, src2[rh:], dst2[rh:])
    pre_a = _gather_half(A, B, src2[:rh], dst2[:rh])
    m_b = _edge_mlp(pre_b, ea2[eh:], w0c, b0r, w1p, b1r, eh)
    s_pb = _scatter_half(m_b, dst2[rh:], z)
    m_a = _edge_mlp(pre_a, ea2[:eh], w0c, b0r, w1p, b1r, 0)
    s_pa = _scatter_half(m_a, dst2[:rh], z)

    return _final(h, s_pa, s_pb,
                  gam_W0[:_DL], gam_W0[_DL:], gam_b0.reshape(1, -1),
                  gam_W1, gam_b1.reshape(1, -1),
                  dec_W0, dec_b0.reshape(1, -1),
                  dec_W1, dec_b1.reshape(1, -1),
                  dec_W2, dec_b2.reshape(1, -1))
